# Initial kernel scaffold; baseline (speedup 1.0000x reference)
import jax
import jax.numpy as jnp
from jax.experimental import pallas as pl


def kernel(x, binpoints, w_lin1, b_lin1, w_conv1, b_conv1, w_conv2, b_conv2, w_conv3, b_conv3, w_asp1, b_asp1, w_asp2, b_asp2, gamma, beta, w_lin2, b_lin2):
    raise NotImplementedError("write your pallas kernel here")


# speedup vs baseline: 1.2520x; 1.2520x over previous
import jax
import jax.numpy as jnp
from jax.experimental import pallas as pl
from jax.experimental.pallas import tpu as pltpu

_NFILT = 80
_NBINS = 257


def _body(x_ref, p_ref, wl1_ref, bl1_ref, wc1_ref, bc1_ref, wc2_ref, bc2_ref,
          wc3_ref, bc3_ref, wa1_ref, ba1_ref, wa2_ref, ba2_ref, g_ref, be_ref,
          wl2_ref, bl2_ref, o_ref):
    f32 = jnp.float32
    bf16 = jnp.bfloat16
    xv = x_ref[0]
    T = xv.shape[0]

    P = p_ref[...]
    bj, bj1, bj2 = P[0:1], P[1:2], P[2:3]
    ibj, ibj1, ibj2 = P[3:4], P[4:5], P[5:6]
    colmask = P[6:7]
    I = jax.lax.broadcasted_iota(jnp.int32, (_NBINS, _NFILT), 0).astype(f32)
    rise_m = (I >= ibj) & (I < ibj1)
    fall_m = (I >= ibj1) & (I < ibj2)
    d1 = (bj1 - bj) ** 2
    d2 = (bj2 - bj1) ** 2
    rise = (I - bj) / jnp.where(d1 > 0, d1, 1.0)
    fall = (bj2 - I) / jnp.where(d2 > 0, d2, 1.0)
    fbT = jnp.where(rise_m, rise, jnp.where(fall_m, fall, 0.0)) * colmask

    filt = jnp.dot(xv.astype(bf16), fbT.astype(bf16),
                   preferred_element_type=f32)
    lane = jax.lax.broadcasted_iota(jnp.int32, (T, _NFILT), 1)
    filt = jnp.where(lane == 0, xv[:, 0:1], filt)
    h = jnp.maximum(
        jnp.dot(filt.astype(bf16), wl1_ref[...], preferred_element_type=f32)
        + bl1_ref[...], 0.0)

    def dconv(hv, w_ref, b_ref, d):
        h16 = hv.astype(bf16)
        cin = h16.shape[1]
        z = jnp.zeros((d, cin), bf16)
        hm = jnp.concatenate([z, h16[:T - d]], axis=0)
        hp = jnp.concatenate([h16[d:], z], axis=0)
        w3 = w_ref[...]
        y = (jnp.dot(hm, w3[0], preferred_element_type=f32)
             + jnp.dot(h16, w3[1], preferred_element_type=f32)
             + jnp.dot(hp, w3[2], preferred_element_type=f32))
        return y + b_ref[...]

    i1 = dconv(h, wc1_ref, bc1_ref, 2)
    i2 = dconv(i1, wc2_ref, bc2_ref, 3) + i1
    i12 = i1 + i2
    i3 = dconv(i12, wc3_ref, bc3_ref, 4) + i12

    a = jnp.tanh(
        jnp.dot(i3.astype(bf16), wa1_ref[...], preferred_element_type=f32)
        + ba1_ref[...])
    e = (jnp.dot(a.astype(bf16), wa2_ref[...], preferred_element_type=f32)
         + ba2_ref[...])
    m = jnp.max(e, axis=0, keepdims=True)
    pexp = jnp.exp(e - m)
    s = jnp.sum(pexp, axis=0, keepdims=True)
    alpha = pexp / s
    mean = jnp.sum(alpha * i3, axis=0, keepdims=True)
    msq = jnp.sum(alpha * i3 * i3, axis=0, keepdims=True)
    std = jnp.sqrt(jnp.clip(msq - mean * mean, 1e-9))
    pooled = jnp.concatenate([mean, std], axis=1)

    mu = jnp.mean(pooled, axis=1, keepdims=True)
    var = jnp.mean((pooled - mu) ** 2, axis=1, keepdims=True)
    gn = (pooled - mu) / jnp.sqrt(var + 1e-5) * g_ref[...] + be_ref[...]
    o_ref[0] = (jnp.dot(gn.astype(bf16), wl2_ref[...],
                        preferred_element_type=f32) + bl2_ref[...])


def kernel(x, binpoints, w_lin1, b_lin1, w_conv1, b_conv1, w_conv2, b_conv2,
           w_conv3, b_conv3, w_asp1, b_asp1, w_asp2, b_asp2, gamma, beta,
           w_lin2, b_lin2):
    f32 = jnp.float32
    bf16 = jnp.bfloat16
    B, T, F = x.shape

    b = binpoints.astype(f32)
    ib = jnp.floor(b)
    P = jnp.stack([
        b[:_NFILT], b[1:_NFILT + 1], b[2:_NFILT + 2],
        ib[:_NFILT], ib[1:_NFILT + 1], ib[2:_NFILT + 2],
        (jnp.arange(_NFILT) < _NFILT - 1).astype(f32),
        jnp.zeros((_NFILT,), f32),
    ])

    wl1t = w_lin1.T.astype(bf16)
    wc1 = jnp.transpose(w_conv1, (2, 1, 0)).astype(bf16)
    wc2 = jnp.transpose(w_conv2, (2, 1, 0)).astype(bf16)
    wc3 = jnp.transpose(w_conv3, (2, 1, 0)).astype(bf16)
    wa1t = w_asp1.T.astype(bf16)
    wa2t = w_asp2.T.astype(bf16)
    wl2t = w_lin2.T.astype(bf16)

    row = lambda v: v.reshape(1, -1)
    full = lambda arr: pl.BlockSpec(arr.shape, lambda i: (0,) * arr.ndim)
    operands = [P, wl1t, row(b_lin1), wc1, row(b_conv1), wc2, row(b_conv2),
                wc3, row(b_conv3), wa1t, row(b_asp1), wa2t, row(b_asp2),
                row(gamma), row(beta), wl2t, row(b_lin2)]

    out = pl.pallas_call(
        _body,
        grid=(B,),
        in_specs=[pl.BlockSpec((1, T, F), lambda i: (i, 0, 0))]
                 + [full(a) for a in operands],
        out_specs=pl.BlockSpec((1, 1, 512), lambda i: (i, 0, 0)),
        out_shape=jax.ShapeDtypeStruct((B, 1, 512), f32),
        compiler_params=pltpu.CompilerParams(
            dimension_semantics=("parallel",),
            vmem_limit_bytes=50 * 1024 * 1024,
        ),
    )(x, *operands)
    return out.reshape(B, 512)


# speedup vs baseline: 1.3710x; 1.0951x over previous
import jax
import jax.numpy as jnp
from jax.experimental import pallas as pl
from jax.experimental.pallas import tpu as pltpu

_NFILT = 80
_NBINS = 257


def _body(x_ref, p_ref, wl1_ref, bl1_ref, wc1_ref, bc1_ref, wc2_ref, bc2_ref,
          wc3_ref, bc3_ref, wa1_ref, ba1_ref, wa2_ref, ba2_ref, g_ref, be_ref,
          wl2_ref, bl2_ref, o_ref):
    f32 = jnp.float32
    bf16 = jnp.bfloat16
    xv = x_ref[0]
    T = xv.shape[0]

    P = p_ref[...]
    bj, bj1, bj2 = P[0:1], P[1:2], P[2:3]
    ibj, ibj1, ibj2 = P[3:4], P[4:5], P[5:6]
    colmask = P[6:7]
    I = jax.lax.broadcasted_iota(jnp.int32, (_NBINS - 1, _NFILT), 0).astype(f32)
    rise_m = (I >= ibj) & (I < ibj1)
    fall_m = (I >= ibj1) & (I < ibj2)
    d1 = (bj1 - bj) ** 2
    d2 = (bj2 - bj1) ** 2
    rise = (I - bj) / jnp.where(d1 > 0, d1, 1.0)
    fall = (bj2 - I) / jnp.where(d2 > 0, d2, 1.0)
    fbT = jnp.where(rise_m, rise, jnp.where(fall_m, fall, 0.0)) * colmask

    filt = jnp.dot(xv[:, :_NBINS - 1], fbT.astype(bf16),
                   preferred_element_type=f32)
    lane = jax.lax.broadcasted_iota(jnp.int32, (T, _NFILT), 1)
    filt = jnp.where(lane == 0, xv[:, 0:1].astype(f32), filt)
    h = jnp.maximum(
        jnp.dot(filt.astype(bf16), wl1_ref[...], preferred_element_type=f32)
        + bl1_ref[...], 0.0)

    def dconv(hv, w_ref, b_ref, d):
        cin = hv.shape[1]
        z = jnp.zeros((d, cin), f32)
        hm = jnp.concatenate([z, hv[:T - d]], axis=0)
        hp = jnp.concatenate([hv[d:], z], axis=0)
        hcat = jnp.concatenate([hm, hv, hp], axis=1).astype(bf16)
        return (jnp.dot(hcat, w_ref[...], preferred_element_type=f32)
                + b_ref[...])

    i1 = dconv(h, wc1_ref, bc1_ref, 2)
    i2 = dconv(i1, wc2_ref, bc2_ref, 3) + i1
    i12 = i1 + i2
    i3 = dconv(i12, wc3_ref, bc3_ref, 4) + i12

    a = jnp.tanh(
        jnp.dot(i3.astype(bf16), wa1_ref[...], preferred_element_type=f32)
        + ba1_ref[...])
    e = (jnp.dot(a.astype(bf16), wa2_ref[...], preferred_element_type=f32)
         + ba2_ref[...])
    m = jnp.max(e, axis=0, keepdims=True)
    pexp = jnp.exp(e - m)
    s = jnp.sum(pexp, axis=0, keepdims=True)
    alpha = pexp / s
    mean = jnp.sum(alpha * i3, axis=0, keepdims=True)
    msq = jnp.sum(alpha * i3 * i3, axis=0, keepdims=True)
    std = jnp.sqrt(jnp.clip(msq - mean * mean, 1e-9))
    pooled = jnp.concatenate([mean, std], axis=1)

    mu = jnp.mean(pooled, axis=1, keepdims=True)
    var = jnp.mean((pooled - mu) ** 2, axis=1, keepdims=True)
    gn = (pooled - mu) / jnp.sqrt(var + 1e-5) * g_ref[...] + be_ref[...]
    o_ref[0] = (jnp.dot(gn.astype(bf16), wl2_ref[...],
                        preferred_element_type=f32) + bl2_ref[...])


def kernel(x, binpoints, w_lin1, b_lin1, w_conv1, b_conv1, w_conv2, b_conv2,
           w_conv3, b_conv3, w_asp1, b_asp1, w_asp2, b_asp2, gamma, beta,
           w_lin2, b_lin2):
    f32 = jnp.float32
    bf16 = jnp.bfloat16
    B, T, F = x.shape

    b = binpoints.astype(f32)
    ib = jnp.floor(b)
    P = jnp.stack([
        b[:_NFILT], b[1:_NFILT + 1], b[2:_NFILT + 2],
        ib[:_NFILT], ib[1:_NFILT + 1], ib[2:_NFILT + 2],
        (jnp.arange(_NFILT) < _NFILT - 1).astype(f32),
        jnp.zeros((_NFILT,), f32),
    ])

    wl1t = w_lin1.T.astype(bf16)
    stack_taps = lambda w: jnp.transpose(w, (2, 1, 0)).reshape(
        3 * w.shape[1], w.shape[0]).astype(bf16)
    wc1 = stack_taps(w_conv1)
    wc2 = stack_taps(w_conv2)
    wc3 = stack_taps(w_conv3)
    wa1t = w_asp1.T.astype(bf16)
    wa2t = w_asp2.T.astype(bf16)
    wl2t = w_lin2.T.astype(bf16)

    row = lambda v: v.reshape(1, -1)
    full = lambda arr: pl.BlockSpec(arr.shape, lambda i: (0,) * arr.ndim)
    operands = [P, wl1t, row(b_lin1), wc1, row(b_conv1), wc2, row(b_conv2),
                wc3, row(b_conv3), wa1t, row(b_asp1), wa2t, row(b_asp2),
                row(gamma), row(beta), wl2t, row(b_lin2)]

    x16 = x.astype(bf16)
    out = pl.pallas_call(
        _body,
        grid=(B,),
        in_specs=[pl.BlockSpec((1, T, F), lambda i: (i, 0, 0))]
                 + [full(a) for a in operands],
        out_specs=pl.BlockSpec((1, 1, 512), lambda i: (i, 0, 0)),
        out_shape=jax.ShapeDtypeStruct((B, 1, 512), f32),
        compiler_params=pltpu.CompilerParams(
            dimension_semantics=("parallel",),
            vmem_limit_bytes=50 * 1024 * 1024,
        ),
    )(x16, *operands)
    return out.reshape(B, 512)


# speedup vs baseline: 3.1285x; 2.2819x over previous
import jax
import jax.numpy as jnp
from jax.experimental import pallas as pl
from jax.experimental.pallas import tpu as pltpu

_NFILT = 80
_NBINS = 257


def _body(x_hbm, p_ref, wl1_ref, bl1_ref, wc1_ref, bc1_ref, wc2_ref, bc2_ref,
          wc3_ref, bc3_ref, wa1_ref, ba1_ref, wa2_ref, ba2_ref, g_ref, be_ref,
          wl2_ref, bl2_ref, o_ref, xbuf, sem):
    f32 = jnp.float32
    bf16 = jnp.bfloat16
    B = x_hbm.shape[1]
    T = x_hbm.shape[2]
    b = pl.program_id(0)
    slot = jax.lax.rem(b, 2)

    def copy_in(bi, si):
        return pltpu.make_async_copy(x_hbm.at[:, bi, :], xbuf.at[si],
                                     sem.at[si])

    @pl.when(b == 0)
    def _():
        copy_in(0, 0).start()

    @pl.when(b + 1 < B)
    def _():
        copy_in(b + 1, jax.lax.rem(b + 1, 2)).start()

    copy_in(b, slot).wait()
    xb = xbuf[slot]

    P = p_ref[...]
    bj, bj1, bj2 = P[:, 0:1], P[:, 1:2], P[:, 2:3]
    ibj, ibj1, ibj2 = P[:, 3:4], P[:, 4:5], P[:, 5:6]
    rowmask = P[:, 6:7]
    I = jax.lax.broadcasted_iota(jnp.int32, (_NFILT, _NBINS - 1), 1).astype(f32)
    rise_m = (I >= ibj) & (I < ibj1)
    fall_m = (I >= ibj1) & (I < ibj2)
    d1 = (bj1 - bj) ** 2
    d2 = (bj2 - bj1) ** 2
    rise = (I - bj) / jnp.where(d1 > 0, d1, 1.0)
    fall = (bj2 - I) / jnp.where(d2 > 0, d2, 1.0)
    fb = jnp.where(rise_m, rise, jnp.where(fall_m, fall, 0.0)) * rowmask

    filt = jnp.dot(fb.astype(bf16), xb[:_NBINS - 1].astype(bf16),
                   preferred_element_type=f32)
    row = jax.lax.broadcasted_iota(jnp.int32, (_NFILT, T), 0)
    filt = jnp.where(row == 0, xb[0:1, :], filt)
    h = jnp.maximum(
        jnp.dot(wl1_ref[...], filt.astype(bf16), preferred_element_type=f32)
        + bl1_ref[...], 0.0)

    def dconv(hv, w_ref, b_ref, d):
        c = hv.shape[0]
        z = jnp.zeros((c, d), f32)
        hm = jnp.concatenate([z, hv[:, :T - d]], axis=1)
        hp = jnp.concatenate([hv[:, d:], z], axis=1)
        hcat = jnp.concatenate([hm, hv, hp], axis=0).astype(bf16)
        return (jnp.dot(w_ref[...], hcat, preferred_element_type=f32)
                + b_ref[...])

    i1 = dconv(h, wc1_ref, bc1_ref, 2)
    i2 = dconv(i1, wc2_ref, bc2_ref, 3) + i1
    i12 = i1 + i2
    i3 = dconv(i12, wc3_ref, bc3_ref, 4) + i12

    a = jnp.tanh(
        jnp.dot(wa1_ref[...], i3.astype(bf16), preferred_element_type=f32)
        + ba1_ref[...])
    e = (jnp.dot(wa2_ref[...], a.astype(bf16), preferred_element_type=f32)
         + ba2_ref[...])
    m = jnp.max(e, axis=1, keepdims=True)
    pexp = jnp.exp(e - m)
    s = jnp.sum(pexp, axis=1, keepdims=True)
    alpha = pexp / s
    mean = jnp.sum(alpha * i3, axis=1, keepdims=True)
    msq = jnp.sum(alpha * i3 * i3, axis=1, keepdims=True)
    std = jnp.sqrt(jnp.clip(msq - mean * mean, 1e-9))
    pooled = jnp.concatenate([mean, std], axis=0)

    mu = jnp.mean(pooled, axis=0, keepdims=True)
    var = jnp.mean((pooled - mu) ** 2, axis=0, keepdims=True)
    gn = (pooled - mu) / jnp.sqrt(var + 1e-5) * g_ref[...] + be_ref[...]
    out = jax.lax.dot_general(gn.astype(bf16), wl2_ref[...],
                              (((0,), (0,)), ((), ())),
                              preferred_element_type=f32)
    o_ref[0] = out + bl2_ref[...]


def kernel(x, binpoints, w_lin1, b_lin1, w_conv1, b_conv1, w_conv2, b_conv2,
           w_conv3, b_conv3, w_asp1, b_asp1, w_asp2, b_asp2, gamma, beta,
           w_lin2, b_lin2):
    f32 = jnp.float32
    bf16 = jnp.bfloat16
    B, T, F = x.shape
    xt = jnp.transpose(x, (2, 0, 1))

    bp = binpoints.astype(f32)
    ib = jnp.floor(bp)
    P = jnp.stack([
        bp[:_NFILT], bp[1:_NFILT + 1], bp[2:_NFILT + 2],
        ib[:_NFILT], ib[1:_NFILT + 1], ib[2:_NFILT + 2],
        (jnp.arange(_NFILT) < _NFILT - 1).astype(f32),
        jnp.zeros((_NFILT,), f32),
    ], axis=1)

    wl1 = w_lin1.astype(bf16)
    stack_taps = lambda w: jnp.transpose(w, (0, 2, 1)).reshape(
        w.shape[0], 3 * w.shape[1]).astype(bf16)
    wc1 = stack_taps(w_conv1)
    wc2 = stack_taps(w_conv2)
    wc3 = stack_taps(w_conv3)
    wa1 = w_asp1.astype(bf16)
    wa2 = w_asp2.astype(bf16)
    wl2t = w_lin2.T.astype(bf16)

    col = lambda v: v.reshape(-1, 1)
    row = lambda v: v.reshape(1, -1)
    full = lambda arr: pl.BlockSpec(arr.shape, lambda i: (0,) * arr.ndim)
    operands = [P, wl1, col(b_lin1), wc1, col(b_conv1), wc2, col(b_conv2),
                wc3, col(b_conv3), wa1, col(b_asp1), wa2, col(b_asp2),
                col(gamma), col(beta), wl2t, row(b_lin2)]

    out = pl.pallas_call(
        _body,
        grid=(B,),
        in_specs=[pl.BlockSpec(memory_space=pl.ANY)]
                 + [full(a) for a in operands],
        out_specs=pl.BlockSpec((1, 1, 512), lambda i: (i, 0, 0)),
        out_shape=jax.ShapeDtypeStruct((B, 1, 512), f32),
        scratch_shapes=[
            pltpu.VMEM((2, F, T), f32),
            pltpu.SemaphoreType.DMA((2,)),
        ],
        compiler_params=pltpu.CompilerParams(
            dimension_semantics=("arbitrary",),
            vmem_limit_bytes=48 * 1024 * 1024,
        ),
    )(xt, *operands)
    return out.reshape(B, 512)


# speedup vs baseline: 3.2404x; 1.0358x over previous
import jax
import jax.numpy as jnp
from jax.experimental import pallas as pl
from jax.experimental.pallas import tpu as pltpu

_NFILT = 80
_NBINS = 257


def _body(x_hbm, p_ref, wl1_ref, bl1_ref, wc1_ref, bc1_ref, wc2_ref, bc2_ref,
          wc3_ref, bc3_ref, wa1_ref, ba1_ref, wa2_ref, ba2_ref, g_ref, be_ref,
          wl2_ref, bl2_ref, o_ref, xbuf, fbbuf, sem):
    f32 = jnp.float32
    bf16 = jnp.bfloat16
    B = x_hbm.shape[1]
    T = x_hbm.shape[2]
    nstep = B // 2
    j = pl.program_id(0)
    spair = jax.lax.rem(j, 2) * 2

    def copy_in(bi, si):
        return pltpu.make_async_copy(x_hbm.at[:, bi, :], xbuf.at[si],
                                     sem.at[si])

    @pl.when(j == 0)
    def _():
        copy_in(0, 0).start()
        copy_in(1, 1).start()

    @pl.when(j + 1 < nstep)
    def _():
        nx = jax.lax.rem(j + 1, 2) * 2
        copy_in(2 * j + 2, nx).start()
        copy_in(2 * j + 3, nx + 1).start()

    @pl.when(j == 0)
    def _():
        P = p_ref[...]
        bj, bj1, bj2 = P[:, 0:1], P[:, 1:2], P[:, 2:3]
        ibj, ibj1, ibj2 = P[:, 3:4], P[:, 4:5], P[:, 5:6]
        rowmask = P[:, 6:7]
        I = jax.lax.broadcasted_iota(
            jnp.int32, (_NFILT, _NBINS - 1), 1).astype(f32)
        rise_m = (I >= ibj) & (I < ibj1)
        fall_m = (I >= ibj1) & (I < ibj2)
        d1 = (bj1 - bj) ** 2
        d2 = (bj2 - bj1) ** 2
        rise = (I - bj) / jnp.where(d1 > 0, d1, 1.0)
        fall = (bj2 - I) / jnp.where(d2 > 0, d2, 1.0)
        fbbuf[...] = (jnp.where(rise_m, rise,
                                jnp.where(fall_m, fall, 0.0))
                      * rowmask).astype(bf16)

    def dconv(hv, w_ref, b_ref, d):
        c = hv.shape[0]
        z = jnp.zeros((c, d), f32)
        hm = jnp.concatenate([z, hv[:, :T - d]], axis=1)
        hp = jnp.concatenate([hv[:, d:], z], axis=1)
        hcat = jnp.concatenate([hm, hv, hp], axis=0).astype(bf16)
        return (jnp.dot(w_ref[...], hcat, preferred_element_type=f32)
                + b_ref[...])

    def one_batch(xb, k):
        filt = jnp.dot(fbbuf[...], xb[:_NBINS - 1].astype(bf16),
                       preferred_element_type=f32)
        row = jax.lax.broadcasted_iota(jnp.int32, (_NFILT, T), 0)
        filt = jnp.where(row == 0, xb[0:1, :], filt)
        h = jnp.maximum(
            jnp.dot(wl1_ref[...], filt.astype(bf16),
                    preferred_element_type=f32) + bl1_ref[...], 0.0)

        i1 = dconv(h, wc1_ref, bc1_ref, 2)
        i2 = dconv(i1, wc2_ref, bc2_ref, 3) + i1
        i12 = i1 + i2
        i3 = dconv(i12, wc3_ref, bc3_ref, 4) + i12

        a = jnp.tanh(
            jnp.dot(wa1_ref[...], i3.astype(bf16), preferred_element_type=f32)
            + ba1_ref[...])
        e = (jnp.dot(wa2_ref[...], a.astype(bf16), preferred_element_type=f32)
             + ba2_ref[...])
        m = jnp.max(e, axis=1, keepdims=True)
        pexp = jnp.exp(e - m)
        s = jnp.sum(pexp, axis=1, keepdims=True)
        alpha = pexp / s
        mean = jnp.sum(alpha * i3, axis=1, keepdims=True)
        msq = jnp.sum(alpha * i3 * i3, axis=1, keepdims=True)
        std = jnp.sqrt(jnp.clip(msq - mean * mean, 1e-9))
        pooled = jnp.concatenate([mean, std], axis=0)

        mu = jnp.mean(pooled, axis=0, keepdims=True)
        var = jnp.mean((pooled - mu) ** 2, axis=0, keepdims=True)
        gn = ((pooled - mu) / jnp.sqrt(var + 1e-5) * g_ref[...]
              + be_ref[...])
        out = jax.lax.dot_general(gn.astype(bf16), wl2_ref[...],
                                  (((0,), (0,)), ((), ())),
                                  preferred_element_type=f32)
        o_ref[0, k] = (out + bl2_ref[...])[0]

    for k in (0, 1):
        copy_in(2 * j + k, spair + k).wait()
        one_batch(xbuf[spair + k], k)


def kernel(x, binpoints, w_lin1, b_lin1, w_conv1, b_conv1, w_conv2, b_conv2,
           w_conv3, b_conv3, w_asp1, b_asp1, w_asp2, b_asp2, gamma, beta,
           w_lin2, b_lin2):
    f32 = jnp.float32
    bf16 = jnp.bfloat16
    B, T, F = x.shape
    xt = jnp.transpose(x, (2, 0, 1))

    bp = binpoints.astype(f32)
    ib = jnp.floor(bp)
    P = jnp.stack([
        bp[:_NFILT], bp[1:_NFILT + 1], bp[2:_NFILT + 2],
        ib[:_NFILT], ib[1:_NFILT + 1], ib[2:_NFILT + 2],
        (jnp.arange(_NFILT) < _NFILT - 1).astype(f32),
        jnp.zeros((_NFILT,), f32),
    ], axis=1)

    wl1 = w_lin1.astype(bf16)
    stack_taps = lambda w: jnp.transpose(w, (0, 2, 1)).reshape(
        w.shape[0], 3 * w.shape[1]).astype(bf16)
    wc1 = stack_taps(w_conv1)
    wc2 = stack_taps(w_conv2)
    wc3 = stack_taps(w_conv3)
    wa1 = w_asp1.astype(bf16)
    wa2 = w_asp2.astype(bf16)
    wl2t = w_lin2.T.astype(bf16)

    col = lambda v: v.reshape(-1, 1)
    row = lambda v: v.reshape(1, -1)
    full = lambda arr: pl.BlockSpec(arr.shape, lambda j: (0,) * arr.ndim)
    operands = [P, wl1, col(b_lin1), wc1, col(b_conv1), wc2, col(b_conv2),
                wc3, col(b_conv3), wa1, col(b_asp1), wa2, col(b_asp2),
                col(gamma), col(beta), wl2t, row(b_lin2)]

    out = pl.pallas_call(
        _body,
        grid=(B // 2,),
        in_specs=[pl.BlockSpec(memory_space=pl.ANY)]
                 + [full(a) for a in operands],
        out_specs=pl.BlockSpec((1, 2, 512), lambda j: (j, 0, 0)),
        out_shape=jax.ShapeDtypeStruct((B // 2, 2, 512), f32),
        scratch_shapes=[
            pltpu.VMEM((4, F, T), f32),
            pltpu.VMEM((_NFILT, _NBINS - 1), bf16),
            pltpu.SemaphoreType.DMA((4,)),
        ],
        compiler_params=pltpu.CompilerParams(
            dimension_semantics=("arbitrary",),
            vmem_limit_bytes=48 * 1024 * 1024,
        ),
    )(xt, *operands)
    return out.reshape(B, 512)
